# Initial kernel scaffold; baseline (speedup 1.0000x reference)
import jax
import jax.numpy as jnp
from jax.experimental import pallas as pl


def kernel(inputs, pos_table):
    raise NotImplementedError("write your pallas kernel here")


# speedup vs baseline: 1.2283x; 1.2283x over previous
import jax
import jax.numpy as jnp
from jax.experimental import pallas as pl


def _add_body(x_ref, t_ref, o_ref):
    o_ref[...] = x_ref[...] + t_ref[...]


def kernel(inputs, pos_table):
    B, S, D = inputs.shape
    blk = 512
    table = pos_table[:S]
    return pl.pallas_call(
        _add_body,
        grid=(S // blk, B),
        in_specs=[
            pl.BlockSpec((1, blk, D), lambda s, b: (b, s, 0)),
            pl.BlockSpec((blk, D), lambda s, b: (s, 0)),
        ],
        out_specs=pl.BlockSpec((1, blk, D), lambda s, b: (b, s, 0)),
        out_shape=jax.ShapeDtypeStruct((B, S, D), jnp.float32),
    )(inputs, table)
